# Initial kernel scaffold; baseline (speedup 1.0000x reference)
import jax
import jax.numpy as jnp
from jax.experimental import pallas as pl


def kernel(x, adj, weight_own, weight_nbr, bias):
    raise NotImplementedError("write your pallas kernel here")


# speedup vs baseline: 1.0018x; 1.0018x over previous
import functools

import jax
import jax.numpy as jnp
from jax.experimental import pallas as pl
from jax.experimental.pallas import tpu as pltpu

N = 10000
DIN = 128
DOUT = 128
BM = 400


def _h_kernel(x_ref, w_ref, h_ref):
    h_ref[...] = jnp.dot(x_ref[...], w_ref[...],
                         preferred_element_type=jnp.float32)


def _main_kernel(adj_ref, h_ref, x_ref, w_own_ref, bias_ref, out_ref):
    nbr = jnp.dot(adj_ref[...], h_ref[...],
                  preferred_element_type=jnp.float32)
    own = jnp.dot(x_ref[...], w_own_ref[...],
                  preferred_element_type=jnp.float32)
    out_ref[...] = nbr + own + bias_ref[...]


@functools.partial(jax.jit, static_argnames=())
def kernel(x, adj, weight_own, weight_nbr, bias):
    h = pl.pallas_call(
        _h_kernel,
        out_shape=jax.ShapeDtypeStruct((N, DOUT), jnp.float32),
    )(x, weight_nbr)

    bias2d = bias.reshape(1, DOUT)
    grid = (N // BM,)
    out = pl.pallas_call(
        _main_kernel,
        grid=grid,
        in_specs=[
            pl.BlockSpec((BM, N), lambda i: (i, 0)),
            pl.BlockSpec((N, DOUT), lambda i: (0, 0)),
            pl.BlockSpec((BM, DIN), lambda i: (i, 0)),
            pl.BlockSpec((DIN, DOUT), lambda i: (0, 0)),
            pl.BlockSpec((1, DOUT), lambda i: (0, 0)),
        ],
        out_specs=pl.BlockSpec((BM, DOUT), lambda i: (i, 0)),
        out_shape=jax.ShapeDtypeStruct((N, DOUT), jnp.float32),
        compiler_params=pltpu.CompilerParams(
            dimension_semantics=("arbitrary",),
        ),
    )(adj, h, x, weight_own, bias2d)
    return out


# speedup vs baseline: 1.0153x; 1.0135x over previous
import functools

import jax
import jax.numpy as jnp
from jax.experimental import pallas as pl
from jax.experimental.pallas import tpu as pltpu

N = 10000
DIN = 128
DOUT = 128
BM = 400


def _h_kernel(x_ref, w_ref, h_ref):
    h_ref[...] = jnp.dot(x_ref[...], w_ref[...],
                         preferred_element_type=jnp.float32).astype(jnp.bfloat16)


def _main_kernel(adj_ref, h_ref, x_ref, w_own_ref, bias_ref, out_ref):
    nbr = jnp.dot(adj_ref[...].astype(jnp.bfloat16), h_ref[...],
                  preferred_element_type=jnp.float32)
    own = jnp.dot(x_ref[...], w_own_ref[...],
                  preferred_element_type=jnp.float32)
    out_ref[...] = nbr + own + bias_ref[...]


@functools.partial(jax.jit, static_argnames=())
def kernel(x, adj, weight_own, weight_nbr, bias):
    h = pl.pallas_call(
        _h_kernel,
        out_shape=jax.ShapeDtypeStruct((N, DOUT), jnp.bfloat16),
    )(x, weight_nbr)

    bias2d = bias.reshape(1, DOUT)
    grid = (N // BM,)
    out = pl.pallas_call(
        _main_kernel,
        grid=grid,
        in_specs=[
            pl.BlockSpec((BM, N), lambda i: (i, 0)),
            pl.BlockSpec((N, DOUT), lambda i: (0, 0)),
            pl.BlockSpec((BM, DIN), lambda i: (i, 0)),
            pl.BlockSpec((DIN, DOUT), lambda i: (0, 0)),
            pl.BlockSpec((1, DOUT), lambda i: (0, 0)),
        ],
        out_specs=pl.BlockSpec((BM, DOUT), lambda i: (i, 0)),
        out_shape=jax.ShapeDtypeStruct((N, DOUT), jnp.float32),
        compiler_params=pltpu.CompilerParams(
            dimension_semantics=("arbitrary",),
        ),
    )(adj, h, x, weight_own, bias2d)
    return out


# speedup vs baseline: 1.0386x; 1.0230x over previous
import functools

import jax
import jax.numpy as jnp
from jax.experimental import pallas as pl
from jax.experimental.pallas import tpu as pltpu

N = 10000
DIN = 128
DOUT = 128
BM = 200
HALF = N // 2


def _h_kernel(x_ref, w_ref, h_ref):
    h_ref[...] = jnp.dot(x_ref[...], w_ref[...],
                         preferred_element_type=jnp.float32).astype(jnp.bfloat16)


def _main_kernel(adj_t_ref, adj_b_ref, h_ref, x_ref, w_own_ref, bias_ref,
                 out_ref):
    h = h_ref[...]
    w_own = w_own_ref[...]
    b = bias_ref[...]
    top = jnp.dot(adj_t_ref[0].astype(jnp.bfloat16), h,
                  preferred_element_type=jnp.float32)
    bot = jnp.dot(adj_b_ref[0].astype(jnp.bfloat16), h,
                  preferred_element_type=jnp.float32)
    own_t = jnp.dot(x_ref[0], w_own, preferred_element_type=jnp.float32)
    own_b = jnp.dot(x_ref[1], w_own, preferred_element_type=jnp.float32)
    out_ref[0] = top + own_t + b
    out_ref[1] = bot + own_b + b


@functools.partial(jax.jit, static_argnames=())
def kernel(x, adj, weight_own, weight_nbr, bias):
    h = pl.pallas_call(
        _h_kernel,
        out_shape=jax.ShapeDtypeStruct((N, DOUT), jnp.bfloat16),
    )(x, weight_nbr)

    adj3 = adj.reshape(2, HALF, N)
    x3 = x.reshape(2, HALF, DIN)
    bias2d = bias.reshape(1, DOUT)
    grid = (HALF // BM,)
    out = pl.pallas_call(
        _main_kernel,
        grid=grid,
        in_specs=[
            pl.BlockSpec((1, BM, N), lambda i: (0, i, 0)),
            pl.BlockSpec((1, BM, N), lambda i: (1, i, 0)),
            pl.BlockSpec((N, DOUT), lambda i: (0, 0)),
            pl.BlockSpec((2, BM, DIN), lambda i: (0, i, 0)),
            pl.BlockSpec((DIN, DOUT), lambda i: (0, 0)),
            pl.BlockSpec((1, DOUT), lambda i: (0, 0)),
        ],
        out_specs=pl.BlockSpec((2, BM, DOUT), lambda i: (0, i, 0)),
        out_shape=jax.ShapeDtypeStruct((2, HALF, DOUT), jnp.float32),
        compiler_params=pltpu.CompilerParams(
            dimension_semantics=("arbitrary",),
        ),
    )(adj3, adj3, h, x3, weight_own, bias2d)
    return out.reshape(N, DOUT)
